# Initial kernel scaffold; baseline (speedup 1.0000x reference)
import functools

import jax
import jax.numpy as jnp
from jax import lax
from jax.experimental import pallas as pl
from jax.experimental.pallas import tpu as pltpu
from jax.experimental.pallas import tpu_sc as plsc

N = 10000
D = 128
E = 320000
NPAD = 10240
NODES_PER_W = 320
NW = 32
BLK = 1600
NBLK = E // BLK
M = 20480
THRESH = M - 2 * BLK - 128
GC = 128
PADEVAL = NODES_PER_W * 16384
NEG = -3.0e38


def _centroid_body(xcen_ref, bcol_ref, brow_sref, wc_ref, bc_ref, wv_ref,
                   z_ref, p_scr, q_scr, xc_scr):
    j = pl.program_id(0)

    @pl.when(j == 0)
    def _init():
        xcen = xcen_ref[:]
        wc1 = wc_ref[0:128, :]
        wc2 = wc_ref[128:256, :]
        p_scr[:] = jnp.dot(xcen, wc1 - wc2,
                           preferred_element_type=jnp.float32) + bc_ref[:]
        q_scr[:] = jnp.dot(xcen, wc2, preferred_element_type=jnp.float32)
        xc_scr[:] = jnp.full((128, 128), NEG, jnp.float32)

    @pl.when(j < 128)
    def _step():
        bj = brow_sref[0, j]
        qrow = q_scr[pl.ds(j, 1), :]
        cand = jnp.maximum(p_scr[:] + qrow, 0.0)
        cand = jnp.where(bcol_ref[:] == bj, cand, NEG)
        xc_scr[:] = jnp.maximum(xc_scr[:], cand)

    @pl.when(j == 128)
    def _fin():
        xc = xc_scr[:]
        xc = jnp.where(xc > NEG * 0.5, xc, 0.0)
        wv3 = wv_ref[256:384, :]
        z_ref[:] = jnp.dot(xc, wv3, preferred_element_type=jnp.float32)


def _centroid_conv(xcen_pad, bcen_col, bcen_row, Wc, bc_row, Wv):
    return pl.pallas_call(
        _centroid_body,
        grid=(129,),
        in_specs=[
            pl.BlockSpec(memory_space=pltpu.VMEM),
            pl.BlockSpec(memory_space=pltpu.VMEM),
            pl.BlockSpec(memory_space=pltpu.SMEM),
            pl.BlockSpec(memory_space=pltpu.VMEM),
            pl.BlockSpec(memory_space=pltpu.VMEM),
            pl.BlockSpec(memory_space=pltpu.VMEM),
        ],
        out_specs=pl.BlockSpec(memory_space=pltpu.VMEM),
        out_shape=jax.ShapeDtypeStruct((128, 128), jnp.float32),
        scratch_shapes=[
            pltpu.VMEM((128, 128), jnp.float32),
            pltpu.VMEM((128, 128), jnp.float32),
            pltpu.VMEM((128, 128), jnp.float32),
        ],
    )(xcen_pad, bcen_col, bcen_row, Wc, bc_row, Wv)


def _tables_body(x_ref, bf_ref, wv_ref, bv_ref, z_ref, a_ref, b_ref):
    xb = x_ref[:]
    wv1 = wv_ref[0:128, :]
    wv2 = wv_ref[128:256, :]
    a_ref[:] = jnp.dot(xb, wv1 - wv2,
                       preferred_element_type=jnp.float32) + bv_ref[:]
    iot = lax.broadcasted_iota(jnp.float32, (1024, 128), 1)
    oh = jnp.where(bf_ref[:] == iot, 1.0, 0.0)
    b_ref[:] = (jnp.dot(xb, wv2, preferred_element_type=jnp.float32)
                + jnp.dot(oh, z_ref[:], preferred_element_type=jnp.float32))


def _node_tables(x_pad, batch_f, Wv, bv_row, z):
    nblk = NPAD // 1024
    return pl.pallas_call(
        _tables_body,
        grid=(nblk,),
        in_specs=[
            pl.BlockSpec((1024, 128), lambda i: (i, 0)),
            pl.BlockSpec((1024, 1), lambda i: (i, 0)),
            pl.BlockSpec((384, 128), lambda i: (0, 0)),
            pl.BlockSpec((1, 128), lambda i: (0, 0)),
            pl.BlockSpec((128, 128), lambda i: (0, 0)),
        ],
        out_specs=[
            pl.BlockSpec((1024, 128), lambda i: (i, 0)),
            pl.BlockSpec((1024, 128), lambda i: (i, 0)),
        ],
        out_shape=[
            jax.ShapeDtypeStruct((NPAD, 128), jnp.float32),
            jax.ShapeDtypeStruct((NPAD, 128), jnp.float32),
        ],
    )(x_pad, batch_f, Wv, bv_row, z)


def _sc_segmax_body(dst_h, src_h, bn_h, out_h,
                    acc, owned, d0, s0, d1, s1, gsrc, gdloc, rows,
                    semA, semB, semG):
    wid = lax.axis_index("s") * 2 + lax.axis_index("c")
    lo = wid * NODES_PER_W
    hi = lo + NODES_PER_W

    def initacc(k, _):
        for c in range(8):
            acc[k, pl.ds(c * 16, 16)] = jnp.full((16,), NEG, jnp.float32)
        return 0

    lax.fori_loop(0, NODES_PER_W + 1, initacc, 0)

    def initown(k, _):
        owned[pl.ds(k * 16, 16)] = jnp.full((16,), PADEVAL, jnp.int32)
        return 0

    lax.fori_loop(0, M // 16, initown, 0)

    def scan_buf(db, sb, ptr):
        def ch(i, ptr):
            dv = db[pl.ds(i * 16, 16)]
            sv = sb[pl.ds(i * 16, 16)]
            m = (dv >= lo) & (dv < hi)
            mi = m.astype(jnp.int32)
            pc = plsc.cumsum(mi)
            ev = jnp.where(m, (dv - lo) * 16384 + sv, PADEVAL)
            plsc.store_scatter(owned, [ptr + pc - 1], ev, mask=m)
            return ptr + jnp.max(pc)

        return lax.fori_loop(0, BLK // 16, ch, ptr)

    def drain_at(start):
        for j in range(8):
            ev = owned[pl.ds(start + j * 16, 16)]
            gsrc[pl.ds(j * 16, 16)] = jnp.bitwise_and(ev, 16383)
            gdloc[pl.ds(j * 16, 16)] = lax.shift_right_logical(ev, 14)
        pltpu.async_copy(bn_h.at[gsrc], rows, semG).wait()

        def eb(e, _):
            d = gdloc[e]
            for c in range(8):
                sl = pl.ds(c * 16, 16)
                acc[d, sl] = jnp.maximum(acc[d, sl], rows[e, sl])
            return 0

        lax.fori_loop(0, GC, eb, 0)

    def drain_round(p):
        start = lax.shift_left(lax.shift_right_logical(p - 1, 7), 7)
        drain_at(start)
        return start

    def drain_while(p):
        return lax.while_loop(lambda q: q >= THRESH, drain_round, p)

    pltpu.async_copy(dst_h.at[pl.ds(0, BLK)], d0, semA)
    pltpu.async_copy(src_h.at[pl.ds(0, BLK)], s0, semA)

    def tb(t, ptr):
        b0 = t * 2
        o0 = b0 * BLK
        o1 = o0 + BLK
        o2 = o0 + 2 * BLK
        pltpu.make_async_copy(dst_h.at[pl.ds(o0, BLK)], d0, semA).wait()
        pltpu.make_async_copy(src_h.at[pl.ds(o0, BLK)], s0, semA).wait()
        pltpu.async_copy(dst_h.at[pl.ds(o1, BLK)], d1, semB)
        pltpu.async_copy(src_h.at[pl.ds(o1, BLK)], s1, semB)
        ptr = scan_buf(d0, s0, ptr)
        ptr = drain_while(ptr)
        pltpu.make_async_copy(dst_h.at[pl.ds(o1, BLK)], d1, semB).wait()
        pltpu.make_async_copy(src_h.at[pl.ds(o1, BLK)], s1, semB).wait()
        pltpu.async_copy(dst_h.at[pl.ds(o2, BLK)], d0, semA)
        pltpu.async_copy(src_h.at[pl.ds(o2, BLK)], s0, semA)
        ptr = scan_buf(d1, s1, ptr)
        ptr = drain_while(ptr)
        return ptr

    ptr = lax.fori_loop(0, NBLK // 2, tb, 0)

    off = NBLK * BLK
    pltpu.make_async_copy(dst_h.at[pl.ds(off, BLK)], d0, semA).wait()
    pltpu.make_async_copy(src_h.at[pl.ds(off, BLK)], s0, semA).wait()

    ptr = lax.while_loop(lambda q: q > 0, drain_round, ptr)

    pltpu.sync_copy(acc.at[pl.ds(0, NODES_PER_W)],
                    out_h.at[pl.ds(lo, NODES_PER_W)])


def _sc_segmax(dst_pad, src_pad, bn):
    mesh = plsc.VectorSubcoreMesh(core_axis_name="c", subcore_axis_name="s")
    k = functools.partial(
        pl.kernel,
        out_type=jax.ShapeDtypeStruct((NPAD, 128), jnp.float32),
        mesh=mesh,
        scratch_types=[
            pltpu.VMEM((NODES_PER_W + 1, 128), jnp.float32),
            pltpu.VMEM((M,), jnp.int32),
            pltpu.VMEM((BLK,), jnp.int32),
            pltpu.VMEM((BLK,), jnp.int32),
            pltpu.VMEM((BLK,), jnp.int32),
            pltpu.VMEM((BLK,), jnp.int32),
            pltpu.VMEM((GC,), jnp.int32),
            pltpu.VMEM((GC,), jnp.int32),
            pltpu.VMEM((GC, 128), jnp.float32),
            pltpu.SemaphoreType.DMA,
            pltpu.SemaphoreType.DMA,
            pltpu.SemaphoreType.DMA,
        ],
    )(_sc_segmax_body)
    return k(dst_pad, src_pad, bn)


def _fin_body(a_ref, acc_ref, o_ref):
    o_ref[:] = jnp.maximum(a_ref[:] + acc_ref[:], 0.0)


def _finalize(a, acc):
    nblk = NPAD // 1024
    return pl.pallas_call(
        _fin_body,
        grid=(nblk,),
        in_specs=[
            pl.BlockSpec((1024, 128), lambda i: (i, 0)),
            pl.BlockSpec((1024, 128), lambda i: (i, 0)),
        ],
        out_specs=pl.BlockSpec((1024, 128), lambda i: (i, 0)),
        out_shape=jax.ShapeDtypeStruct((NPAD, 128), jnp.float32),
    )(a, acc)


def kernel(x, batch, edge_index, x_center, batch_center, Wc, bc, Wv, bv):
    x_pad = jnp.pad(x, ((0, NPAD - N), (0, 0)))
    batch_f = jnp.pad(batch.astype(jnp.float32), (0, NPAD - N))[:, None]
    ei = edge_index.astype(jnp.int32)
    src_pad = jnp.pad(ei[0], (0, 2 * BLK))
    dst_pad = jnp.pad(ei[1], (0, 2 * BLK), constant_values=NPAD)
    xcen_pad = jnp.pad(x_center, ((0, 28), (0, 0)))
    bcen = jnp.pad(batch_center.astype(jnp.float32), (0, 28),
                   constant_values=-1.0)
    bcen_col = bcen[:, None]
    bcen_row = bcen[None, :]
    bc_row = bc[None, :]
    bv_row = bv[None, :]

    z = _centroid_conv(xcen_pad, bcen_col, bcen_row, Wc, bc_row, Wv)
    a_tab, b_tab = _node_tables(x_pad, batch_f, Wv, bv_row, z)
    acc = _sc_segmax(dst_pad, src_pad, b_tab)
    out = _finalize(a_tab, acc)
    return out[:N]


# speedup vs baseline: 5.2311x; 5.2311x over previous
import functools

import jax
import jax.numpy as jnp
from jax import lax
from jax.experimental import pallas as pl
from jax.experimental.pallas import tpu as pltpu
from jax.experimental.pallas import tpu_sc as plsc

N = 10000
D = 128
E = 320000
NPAD = 10240
NODES_PER_W = 320
NW = 32
BLK = 1600
NBLK = E // BLK
M = 20480
THRESH = M - 2 * BLK - 128
GC = 128
PADEVAL = NODES_PER_W * 16384
NEG = -3.0e38


def _centroid_body(xcen_ref, bcol_ref, brow_sref, wc_ref, bc_ref, wv_ref,
                   z_ref, p_scr, q_scr, xc_scr):
    j = pl.program_id(0)

    @pl.when(j == 0)
    def _init():
        xcen = xcen_ref[:]
        wc1 = wc_ref[0:128, :]
        wc2 = wc_ref[128:256, :]
        p_scr[:] = jnp.dot(xcen, wc1 - wc2,
                           preferred_element_type=jnp.float32) + bc_ref[:]
        q_scr[:] = jnp.dot(xcen, wc2, preferred_element_type=jnp.float32)
        xc_scr[:] = jnp.full((128, 128), NEG, jnp.float32)

    @pl.when(j < 128)
    def _step():
        bj = brow_sref[0, j]
        qrow = q_scr[pl.ds(j, 1), :]
        cand = jnp.maximum(p_scr[:] + qrow, 0.0)
        cand = jnp.where(bcol_ref[:] == bj, cand, NEG)
        xc_scr[:] = jnp.maximum(xc_scr[:], cand)

    @pl.when(j == 128)
    def _fin():
        xc = xc_scr[:]
        xc = jnp.where(xc > NEG * 0.5, xc, 0.0)
        wv3 = wv_ref[256:384, :]
        z_ref[:] = jnp.dot(xc, wv3, preferred_element_type=jnp.float32)


def _centroid_conv(xcen_pad, bcen_col, bcen_row, Wc, bc_row, Wv):
    return pl.pallas_call(
        _centroid_body,
        grid=(129,),
        in_specs=[
            pl.BlockSpec(memory_space=pltpu.VMEM),
            pl.BlockSpec(memory_space=pltpu.VMEM),
            pl.BlockSpec(memory_space=pltpu.SMEM),
            pl.BlockSpec(memory_space=pltpu.VMEM),
            pl.BlockSpec(memory_space=pltpu.VMEM),
            pl.BlockSpec(memory_space=pltpu.VMEM),
        ],
        out_specs=pl.BlockSpec(memory_space=pltpu.VMEM),
        out_shape=jax.ShapeDtypeStruct((128, 128), jnp.float32),
        scratch_shapes=[
            pltpu.VMEM((128, 128), jnp.float32),
            pltpu.VMEM((128, 128), jnp.float32),
            pltpu.VMEM((128, 128), jnp.float32),
        ],
    )(xcen_pad, bcen_col, bcen_row, Wc, bc_row, Wv)


def _tables_body(x_ref, bf_ref, wv_ref, bv_ref, z_ref, a_ref, b_ref):
    xb = x_ref[:]
    wv1 = wv_ref[0:128, :]
    wv2 = wv_ref[128:256, :]
    a_ref[:] = jnp.dot(xb, wv1 - wv2,
                       preferred_element_type=jnp.float32) + bv_ref[:]
    iot = lax.broadcasted_iota(jnp.int32, (1024, 128), 1).astype(jnp.float32)
    oh = jnp.where(bf_ref[:] == iot, 1.0, 0.0)
    b_ref[:] = (jnp.dot(xb, wv2, preferred_element_type=jnp.float32)
                + jnp.dot(oh, z_ref[:], preferred_element_type=jnp.float32))


def _node_tables(x_pad, batch_f, Wv, bv_row, z):
    nblk = NPAD // 1024
    return pl.pallas_call(
        _tables_body,
        grid=(nblk,),
        in_specs=[
            pl.BlockSpec((1024, 128), lambda i: (i, 0)),
            pl.BlockSpec((1024, 1), lambda i: (i, 0)),
            pl.BlockSpec((384, 128), lambda i: (0, 0)),
            pl.BlockSpec((1, 128), lambda i: (0, 0)),
            pl.BlockSpec((128, 128), lambda i: (0, 0)),
        ],
        out_specs=[
            pl.BlockSpec((1024, 128), lambda i: (i, 0)),
            pl.BlockSpec((1024, 128), lambda i: (i, 0)),
        ],
        out_shape=[
            jax.ShapeDtypeStruct((NPAD, 128), jnp.float32),
            jax.ShapeDtypeStruct((NPAD, 128), jnp.float32),
        ],
    )(x_pad, batch_f, Wv, bv_row, z)


def _sc_segmax_body(dst_h, src_h, bn_h, out_h,
                    acc, owned, d0, s0, d1, s1,
                    gsrc0, gdloc0, rows0, gsrc1, gdloc1, rows1,
                    semA, semB, semG0, semG1):
    wid = lax.axis_index("s") * 2 + lax.axis_index("c")
    lo = wid * NODES_PER_W
    hi = lo + NODES_PER_W

    def initacc(k, _):
        for c in range(8):
            acc[k, pl.ds(c * 16, 16)] = jnp.full((16,), NEG, jnp.float32)
        return 0

    lax.fori_loop(0, NODES_PER_W + 1, initacc, 0)

    def initown(k, _):
        owned[pl.ds(k * 16, 16)] = jnp.full((16,), PADEVAL, jnp.int32)
        return 0

    lax.fori_loop(0, M // 16, initown, 0)

    def scan_buf(db, sb, ptr):
        def ch(i, ptr):
            dv = db[pl.ds(i * 16, 16)]
            sv = sb[pl.ds(i * 16, 16)]
            m = (dv >= lo) & (dv < hi)
            ev = (dv - lo) * 16384 + sv
            plsc.store_compressed(owned.at[pl.ds(ptr, 16)], ev, mask=m)
            cnt = plsc.all_reduce_population_count(m)[0]
            return ptr + cnt

        return lax.fori_loop(0, BLK // 16, ch, ptr)

    def decode(start, gs, gd):
        for j in range(8):
            ev = owned[pl.ds(start + j * 16, 16)]
            gs[pl.ds(j * 16, 16)] = jnp.bitwise_and(ev, 16383)
            gd[pl.ds(j * 16, 16)] = lax.shift_right_logical(ev, 14)

    def rmw(gd, ro):
        def gb(g, _):
            dvec = gd[pl.ds(g * 16, 16)]
            for l in range(16):
                d = dvec[l]
                e = g * 16 + l
                for c in range(8):
                    sl = pl.ds(c * 16, 16)
                    acc[d, sl] = jnp.maximum(acc[d, sl], ro[e, sl])
            return 0

        lax.fori_loop(0, GC // 16, gb, 0)

    def drain_at(start):
        decode(start, gsrc0, gdloc0)
        pltpu.async_copy(bn_h.at[gsrc0], rows0, semG0).wait()
        rmw(gdloc0, rows0)

    def drain_round(p):
        start = lax.shift_left(lax.shift_right_logical(p - 1, 7), 7)
        drain_at(start)
        return start

    def drain_while(p):
        return lax.while_loop(lambda q: q >= THRESH, drain_round, p)

    pltpu.async_copy(dst_h.at[pl.ds(0, BLK)], d0, semA)
    pltpu.async_copy(src_h.at[pl.ds(0, BLK)], s0, semA)

    def tb(t, ptr):
        b0 = t * 2
        o0 = b0 * BLK
        o1 = o0 + BLK
        o2 = o0 + 2 * BLK
        pltpu.make_async_copy(dst_h.at[pl.ds(o0, BLK)], d0, semA).wait()
        pltpu.make_async_copy(src_h.at[pl.ds(o0, BLK)], s0, semA).wait()
        pltpu.async_copy(dst_h.at[pl.ds(o1, BLK)], d1, semB)
        pltpu.async_copy(src_h.at[pl.ds(o1, BLK)], s1, semB)
        ptr = scan_buf(d0, s0, ptr)
        ptr = drain_while(ptr)
        pltpu.make_async_copy(dst_h.at[pl.ds(o1, BLK)], d1, semB).wait()
        pltpu.make_async_copy(src_h.at[pl.ds(o1, BLK)], s1, semB).wait()
        pltpu.async_copy(dst_h.at[pl.ds(o2, BLK)], d0, semA)
        pltpu.async_copy(src_h.at[pl.ds(o2, BLK)], s0, semA)
        ptr = scan_buf(d1, s1, ptr)
        ptr = drain_while(ptr)
        return ptr

    ptr = lax.fori_loop(0, NBLK // 2, tb, 0)

    off = NBLK * BLK
    pltpu.make_async_copy(dst_h.at[pl.ds(off, BLK)], d0, semA).wait()
    pltpu.make_async_copy(src_h.at[pl.ds(off, BLK)], s0, semA).wait()

    nr = lax.div(ptr + (GC - 1), GC)

    @pl.when(nr > 0)
    def _prime_drain():
        decode(0, gsrc0, gdloc0)
        pltpu.async_copy(bn_h.at[gsrc0], rows0, semG0)

    def pipe_step(r, gs, gd, ro, semX, gs2, gd2, ro2, semY):
        @pl.when(r + 1 < nr)
        def _pf():
            decode((r + 1) * GC, gs2, gd2)
            pltpu.async_copy(bn_h.at[gs2], ro2, semY)

        pltpu.make_async_copy(bn_h.at[gs], ro, semX).wait()
        rmw(gd, ro)

    def rr(r, _):
        even = lax.rem(r, 2) == 0

        @pl.when(even)
        def _e():
            pipe_step(r, gsrc0, gdloc0, rows0, semG0,
                      gsrc1, gdloc1, rows1, semG1)

        @pl.when(jnp.logical_not(even))
        def _o():
            pipe_step(r, gsrc1, gdloc1, rows1, semG1,
                      gsrc0, gdloc0, rows0, semG0)

        return 0

    lax.fori_loop(0, nr, rr, 0)

    pltpu.sync_copy(acc.at[pl.ds(0, NODES_PER_W)],
                    out_h.at[pl.ds(lo, NODES_PER_W)])


def _sc_segmax(dst_pad, src_pad, bn):
    mesh = plsc.VectorSubcoreMesh(core_axis_name="c", subcore_axis_name="s")
    k = functools.partial(
        pl.kernel,
        compiler_params=pltpu.CompilerParams(needs_layout_passes=False),
        out_type=jax.ShapeDtypeStruct((NPAD, 128), jnp.float32),
        mesh=mesh,
        scratch_types=[
            pltpu.VMEM((NODES_PER_W + 1, 128), jnp.float32),
            pltpu.VMEM((M,), jnp.int32),
            pltpu.VMEM((BLK,), jnp.int32),
            pltpu.VMEM((BLK,), jnp.int32),
            pltpu.VMEM((BLK,), jnp.int32),
            pltpu.VMEM((BLK,), jnp.int32),
            pltpu.VMEM((GC,), jnp.int32),
            pltpu.VMEM((GC + 16,), jnp.int32),
            pltpu.VMEM((GC, 128), jnp.float32),
            pltpu.VMEM((GC,), jnp.int32),
            pltpu.VMEM((GC + 16,), jnp.int32),
            pltpu.VMEM((GC, 128), jnp.float32),
            pltpu.SemaphoreType.DMA,
            pltpu.SemaphoreType.DMA,
            pltpu.SemaphoreType.DMA,
            pltpu.SemaphoreType.DMA,
        ],
    )(_sc_segmax_body)
    return k(dst_pad, src_pad, bn)


def _fin_body(a_ref, acc_ref, o_ref):
    o_ref[:] = jnp.maximum(a_ref[:] + acc_ref[:], 0.0)


def _finalize(a, acc):
    nblk = NPAD // 1024
    return pl.pallas_call(
        _fin_body,
        grid=(nblk,),
        in_specs=[
            pl.BlockSpec((1024, 128), lambda i: (i, 0)),
            pl.BlockSpec((1024, 128), lambda i: (i, 0)),
        ],
        out_specs=pl.BlockSpec((1024, 128), lambda i: (i, 0)),
        out_shape=jax.ShapeDtypeStruct((NPAD, 128), jnp.float32),
    )(a, acc)


def kernel(x, batch, edge_index, x_center, batch_center, Wc, bc, Wv, bv):
    x_pad = jnp.pad(x, ((0, NPAD - N), (0, 0)))
    batch_f = jnp.pad(batch.astype(jnp.float32), (0, NPAD - N))[:, None]
    ei = edge_index.astype(jnp.int32)
    src_pad = jnp.pad(ei[0], (0, 2 * BLK))
    dst_pad = jnp.pad(ei[1], (0, 2 * BLK), constant_values=NPAD)
    xcen_pad = jnp.pad(x_center, ((0, 28), (0, 0)))
    bcen = jnp.pad(batch_center.astype(jnp.float32), (0, 28),
                   constant_values=-1.0)
    bcen_col = bcen[:, None]
    bcen_row = bcen[None, :]
    bc_row = bc[None, :]
    bv_row = bv[None, :]

    z = _centroid_conv(xcen_pad, bcen_col, bcen_row, Wc, bc_row, Wv)
    a_tab, b_tab = _node_tables(x_pad, batch_f, Wv, bv_row, z)
    acc = _sc_segmax(dst_pad, src_pad, b_tab)
    out = _finalize(a_tab, acc)
    return out[:N]
